# baseline (speedup 1.0000x reference)
import functools

import jax
import jax.numpy as jnp
from jax import lax
from jax.experimental import pallas as pl
from jax.experimental.pallas import tpu as pltpu
from jax.experimental.pallas import tpu_sc as plsc

B, S, C, W, H = 16, 512, 1, 28, 28
NC, P, NS, HID = 10, 2, 8, 64
D = C * W * H
NQ = HID // 16
K4 = S // 128
SHALF = S // 2


def _dense_kernel(x_ref, w4_ref, bbb_ref, wc1T_ref, bc1_ref, wc2T_ref,
                  bc2_ref, wd1pT_ref, bd1_ref, logp_ref, prop_ref, pred_ref):
    xb = x_ref[0]
    lg_all = jnp.dot(w4_ref[...], xb,
                     preferred_element_type=jnp.float32)
    zpad = jnp.zeros((128, 128 - HID), jnp.float32)
    for k in range(K4):
        lg = lg_all[k * NC:(k + 1) * NC, :] + bbb_ref[...]
        lmax = jnp.max(lg, axis=0, keepdims=True)
        lse = lmax + jnp.log(jnp.sum(jnp.exp(lg - lmax), axis=0, keepdims=True))
        logp = lg - lse
        probs = jnp.exp(logp)
        h = jnp.maximum(jnp.dot(wc1T_ref[...], probs,
                                preferred_element_type=jnp.float32)
                        + bc1_ref[...], 0.0)
        t = jnp.dot(wc2T_ref[...], h,
                    preferred_element_type=jnp.float32) + bc2_ref[...]
        prop = 1.0 / (1.0 + jnp.exp(-t))
        pred = jnp.dot(wd1pT_ref[...], prop,
                       preferred_element_type=jnp.float32) + bd1_ref[...]
        logp_ref[0, :, pl.ds(k * 128, 128)] = logp
        prop_ref[0, :, pl.ds(k * 128, 128)] = prop
        predT = jnp.concatenate([jnp.transpose(pred, (1, 0)), zpad], axis=1)
        pred_ref[0, pl.ds(k * 128, 128), :] = predT


def _sc_scan(pre_hbm, len_hbm, w1_hbm, w2t_hbm, bd2_hbm,
             zout_hbm,
             pre_v, w1_v, w2t_v, bd2_v, len_v, zbuf_v):
    cid = lax.axis_index("c")
    sid = lax.axis_index("s")

    @pl.when(sid < 8)
    def _():
        b = cid * 8 + sid
        pltpu.sync_copy(pre_hbm.at[b], pre_v)
        pltpu.sync_copy(w1_hbm, w1_v)
        pltpu.sync_copy(w2t_hbm, w2t_v)
        pltpu.sync_copy(bd2_hbm, bd2_v)
        pltpu.sync_copy(len_hbm, len_v)

        lane = lax.iota(jnp.int32, 16)
        s0 = jnp.where(lane == 0, 1.0, 0.0).astype(jnp.float32)
        bd2 = bd2_v[...]
        w1r = [[w1_v[pl.ds(j * HID + q * 16, 16)] for q in range(NQ)]
               for j in range(NS)]
        w2r = [[w2t_v[pl.ds(n * HID + q * 16, 16)] for q in range(NQ)]
               for n in range(NS)]

        gdn = lax.GatherDimensionNumbers(
            offset_dims=(), collapsed_slice_dims=(0,), start_index_map=(0,))

        def _bcast(v, j):
            return lax.gather(v, jnp.full((16, 1), j, jnp.int32), gdn,
                              slice_sizes=(1,),
                              mode=lax.GatherScatterMode.PROMISE_IN_BOUNDS)

        L = _bcast(len_v[...], b)[0]

        onehot = [jnp.where(lane == n, 1.0, 0.0).astype(jnp.float32)
                  for n in range(NS)]

        def _tree_add(vs):
            while len(vs) > 1:
                vs = [vs[i] + vs[i + 1] for i in range(0, len(vs) - 1, 2)] + (
                    [vs[-1]] if len(vs) % 2 else [])
            return vs[0]

        carry = (s0, jnp.ones((16,), jnp.float32), jnp.zeros((16,), jnp.float32))
        for seg in range(2):
            lo = seg * SHALF
            hi = lo + SHALF

            def body(t, carry, lo=lo):
                ez, rinv, _ = carry
                ej = [_bcast(ez, j) for j in range(NS)]
                h = []
                for q in range(NQ):
                    acc = _tree_add([ej[j] * w1r[j][q] for j in range(NS)])
                    c = pre_v[t, pl.ds(q * 16, 16)]
                    h.append(jnp.maximum(acc * rinv + c, 0.0))
                zparts = []
                for n in range(NS):
                    v = (h[0] * w2r[n][0] + h[1] * w2r[n][1]) + (
                        h[2] * w2r[n][2] + h[3] * w2r[n][3])
                    zparts.append(jnp.sum(v) * onehot[n])
                z = bd2 + _tree_add(zparts)
                ez2 = jnp.exp(z)
                se = jnp.sum(ez2)
                sev = jnp.broadcast_to(se, (16,))
                rinv2 = jnp.ones((16,), jnp.float32) / sev
                zst = jnp.where(lane == NS, se, z)
                zbuf_v[t - lo, pl.ds(0, 16)] = zst
                return ez2, rinv2, zst

            lm = jnp.clip(L, lo, hi)
            carry = lax.fori_loop(lo, lm, body, carry)
            zlast = carry[2]

            def tail(t, c, lo=lo):
                zbuf_v[t - lo, pl.ds(0, 16)] = zlast
                return c

            lax.fori_loop(lm, hi, tail, 0)
            pltpu.sync_copy(zbuf_v, zout_hbm.at[b, pl.ds(lo, SHALF), :])


def _post_kernel(zraw_ref, mf_ref, logp_ref, prop_ref,
                 ls_ref, label_ref, vars_ref, props_ref):
    zraw = zraw_ref[...]
    logse = jnp.log(zraw[:, :, NS:NS + 1])
    lsq = zraw[:, :, :NS] - logse
    ls_ref[:, :, 1:] = jnp.transpose(lsq, (0, 2, 1))
    mid = jax.lax.broadcasted_iota(jnp.int32, (B, NS, 1), 1)
    ls_ref[:, :, 0:1] = jnp.where(mid == 0, 0.0, -jnp.inf).astype(jnp.float32)
    label_ref[...] = jnp.exp(
        zraw[:, S - 1, NS - 1:NS] - jnp.log(zraw[:, S - 1, NS:NS + 1]))

    mf = mf_ref[...]
    mnext = jnp.concatenate([mf[:, 1:], jnp.zeros((B, 1), jnp.float32)],
                            axis=1)
    d = mf - mnext
    m2 = mf > 0.0
    for c in range(NC):
        lp = logp_ref[:, c, :]
        last = jnp.sum(d * lp, axis=1, keepdims=True)
        vars_ref[c] = jnp.where(m2, lp, jnp.broadcast_to(last, (B, S)))
    for c in range(P):
        pp = prop_ref[:, c, :]
        last = jnp.sum(d * pp, axis=1, keepdims=True)
        props_ref[c] = jnp.where(m2, pp, jnp.broadcast_to(last, (B, S)))


@jax.jit
def kernel(imgs, mask, W_bb, b_bb, W_c1, b_c1, W_c2, b_c2, W_d1, b_d1, W_d2, b_d2):
    x4 = jnp.transpose(imgs.reshape(B, S, D), (0, 2, 1)).reshape(B, D * K4, 128)
    w4 = jnp.zeros((K4, NC, D, K4), jnp.float32)
    for k in range(K4):
        w4 = w4.at[k, :, :, k].set(W_bb.T)
    w4 = w4.reshape(K4 * NC, D * K4)

    logp_t, prop_t, pred_t = pl.pallas_call(
        _dense_kernel,
        grid=(B,),
        in_specs=[
            pl.BlockSpec((1, D * K4, 128), lambda i: (i, 0, 0)),
            pl.BlockSpec((K4 * NC, D * K4), lambda i: (0, 0)),
            pl.BlockSpec((NC, 1), lambda i: (0, 0)),
            pl.BlockSpec((HID, NC), lambda i: (0, 0)),
            pl.BlockSpec((HID, 1), lambda i: (0, 0)),
            pl.BlockSpec((P, HID), lambda i: (0, 0)),
            pl.BlockSpec((P, 1), lambda i: (0, 0)),
            pl.BlockSpec((HID, P), lambda i: (0, 0)),
            pl.BlockSpec((HID, 1), lambda i: (0, 0)),
        ],
        out_specs=[
            pl.BlockSpec((1, NC, S), lambda i: (i, 0, 0)),
            pl.BlockSpec((1, P, S), lambda i: (i, 0, 0)),
            pl.BlockSpec((1, S, 128), lambda i: (i, 0, 0)),
        ],
        out_shape=[
            jax.ShapeDtypeStruct((B, NC, S), jnp.float32),
            jax.ShapeDtypeStruct((B, P, S), jnp.float32),
            jax.ShapeDtypeStruct((B, S, 128), jnp.float32),
        ],
    )(x4, w4, b_bb.reshape(NC, 1), W_c1.T, b_c1.reshape(HID, 1),
      W_c2.T, b_c2.reshape(P, 1), W_d1[NS:].T, b_d1.reshape(HID, 1))

    lengths = jnp.sum(mask, axis=1).astype(jnp.int32)
    bd2_pad = jnp.concatenate(
        [b_d2.astype(jnp.float32), jnp.full((16 - NS,), -jnp.inf, jnp.float32)])

    sc_fn = functools.partial(
        pl.kernel,
        out_type=jax.ShapeDtypeStruct((B, S, 128), jnp.float32),
        mesh=plsc.VectorSubcoreMesh(core_axis_name="c", subcore_axis_name="s"),
        compiler_params=pltpu.CompilerParams(needs_layout_passes=False),
        scratch_types=[
            pltpu.VMEM((S, 128), jnp.float32),
            pltpu.VMEM((NS * HID,), jnp.float32),
            pltpu.VMEM((NS * HID,), jnp.float32),
            pltpu.VMEM((16,), jnp.float32),
            pltpu.VMEM((B,), jnp.int32),
            pltpu.VMEM((SHALF, 128), jnp.float32),
        ],
    )(_sc_scan)
    zraw = sc_fn(pred_t, lengths,
                 W_d1[:NS].reshape(-1), W_d2.T.reshape(-1), bd2_pad)

    mf = mask.astype(jnp.float32)
    ls_t, label2, vars_t, props_t = pl.pallas_call(
        _post_kernel,
        out_shape=[
            jax.ShapeDtypeStruct((B, NS, S + 1), jnp.float32),
            jax.ShapeDtypeStruct((B, 1), jnp.float32),
            jax.ShapeDtypeStruct((NC, B, S), jnp.float32),
            jax.ShapeDtypeStruct((P, B, S), jnp.float32),
        ],
    )(zraw, mf, logp_t, prop_t)

    vars_out = jnp.transpose(vars_t, (1, 2, 0))
    props_out = jnp.transpose(props_t, (1, 2, 0))
    log_states = jnp.transpose(ls_t, (0, 2, 1))
    return (vars_out, props_out, log_states, label2.reshape(B))


# speedup vs baseline: 1.4272x; 1.4272x over previous
import functools

import jax
import jax.numpy as jnp
from jax import lax
from jax.experimental import pallas as pl
from jax.experimental.pallas import tpu as pltpu
from jax.experimental.pallas import tpu_sc as plsc

B, S, C, W, H = 16, 512, 1, 28, 28
NC, P, NS, HID = 10, 2, 8, 64
D = C * W * H
NQ = HID // 16

_HI = jax.lax.Precision.DEFAULT


def _dense_kernel(x_ref, wbb_ref, bbb_ref, wc1_ref, bc1_ref, wc2_ref, bc2_ref,
                  wd1p_ref, bd1_ref, logp_ref, prop_ref, pred_ref):
    x = x_ref[...]
    logits = jnp.dot(x, wbb_ref[...], preferred_element_type=jnp.float32,
                     precision=_HI) + bbb_ref[...]
    lmax = jnp.max(logits, axis=-1, keepdims=True)
    lse = lmax + jnp.log(jnp.sum(jnp.exp(logits - lmax), axis=-1, keepdims=True))
    logp = logits - lse
    probs = jnp.exp(logp)
    h = jnp.maximum(jnp.dot(probs, wc1_ref[...], preferred_element_type=jnp.float32,
                            precision=_HI) + bc1_ref[...], 0.0)
    t = jnp.dot(h, wc2_ref[...], preferred_element_type=jnp.float32,
                precision=_HI) + bc2_ref[...]
    prop = 1.0 / (1.0 + jnp.exp(-t))
    pred = jnp.dot(prop, wd1p_ref[...], preferred_element_type=jnp.float32,
                   precision=_HI) + bd1_ref[...]
    logp_ref[...] = logp
    prop_ref[...] = prop
    pred_ref[...] = pred


def _sc_scan(pre_hbm, len_hbm, w1_hbm, w2t_hbm, bd2_hbm,
             zout_hbm,
             pre_v, w1_v, w2t_v, bd2_v, len_v, zbuf_v):
    cid = lax.axis_index("c")
    sid = lax.axis_index("s")

    @pl.when(sid < 8)
    def _():
        b = cid * 8 + sid
        pltpu.sync_copy(pre_hbm.at[b], pre_v)
        pltpu.sync_copy(w1_hbm, w1_v)
        pltpu.sync_copy(w2t_hbm, w2t_v)
        pltpu.sync_copy(bd2_hbm, bd2_v)
        pltpu.sync_copy(len_hbm, len_v)

        lane = lax.iota(jnp.int32, 16)
        s0 = jnp.where(lane == 0, 1.0, 0.0).astype(jnp.float32)
        bd2 = bd2_v[...]
        w1r = [[w1_v[pl.ds(j * HID + q * 16, 16)] for q in range(NQ)] for j in range(NS)]
        w2r = [[w2t_v[pl.ds(n * HID + q * 16, 16)] for q in range(NQ)] for n in range(NS)]

        gdn = lax.GatherDimensionNumbers(
            offset_dims=(), collapsed_slice_dims=(0,), start_index_map=(0,))

        def _bcast(v, j):
            return lax.gather(v, jnp.full((16, 1), j, jnp.int32), gdn,
                              slice_sizes=(1,),
                              mode=lax.GatherScatterMode.PROMISE_IN_BOUNDS)

        L = _bcast(len_v[...], b)[0]

        onehot = [jnp.where(lane == n, 1.0, 0.0).astype(jnp.float32)
                  for n in range(NS)]

        def _tree_add(vs):
            while len(vs) > 1:
                vs = [vs[i] + vs[i + 1] for i in range(0, len(vs) - 1, 2)] + (
                    [vs[-1]] if len(vs) % 2 else [])
            return vs[0]

        def body(t, carry):
            ez, rinv = carry
            ej = [_bcast(ez, j) for j in range(NS)]
            h = []
            for q in range(NQ):
                acc = _tree_add([ej[j] * w1r[j][q] for j in range(NS)])
                h.append(jnp.maximum(
                    acc * rinv + pre_v[pl.ds(t * HID + q * 16, 16)], 0.0))
            zparts = []
            for n in range(NS):
                v = (h[0] * w2r[n][0] + h[1] * w2r[n][1]) + (
                    h[2] * w2r[n][2] + h[3] * w2r[n][3])
                zparts.append(jnp.sum(v) * onehot[n])
            z = bd2 + _tree_add(zparts)
            ez2 = jnp.exp(z)
            se = jnp.sum(ez2)
            sev = jnp.broadcast_to(se, (16,))
            rinv2 = jnp.ones((16,), jnp.float32) / sev
            zbuf_v[t, pl.ds(0, 16)] = jnp.where(lane == NS, se, z)
            return ez2, rinv2

        lax.fori_loop(0, L, body, (s0, jnp.ones((16,), jnp.float32)))

        zlast = zbuf_v[L - 1, pl.ds(0, 16)]

        def tail(t, carry):
            zbuf_v[t, pl.ds(0, 16)] = zlast
            return carry

        lax.fori_loop(L, S, tail, 0)

        pltpu.sync_copy(zbuf_v, zout_hbm.at[b])


def _post_kernel(zraw_ref, mf_ref, logp_ref, prop_ref,
                 ls_ref, label_ref, vars_ref, props_ref):
    zraw = zraw_ref[...]
    zs = zraw[:, :, :NS]
    lsq = zs - jnp.log(zraw[:, :, NS:NS + 1])
    col = jax.lax.broadcasted_iota(jnp.int32, (B, 1, NS), 2)
    ls_ref[:, 0:1, :] = jnp.where(col == 0, 0.0, -jnp.inf).astype(jnp.float32)
    ls_ref[:, 1:, :] = lsq
    label_ref[...] = jnp.exp(lsq[:, S - 1, NS - 1:NS])

    mf = mf_ref[...]
    mnext = jnp.concatenate([mf[:, 1:], jnp.zeros((B, 1), jnp.float32)], axis=1)
    d = (mf - mnext)[:, :, None]
    m3 = mf[:, :, None] > 0.0
    logp = logp_ref[...]
    last_var = jnp.sum(d * logp, axis=1, keepdims=True)
    vars_ref[...] = jnp.where(m3, logp, jnp.broadcast_to(last_var, (B, S, NC)))
    prop = prop_ref[...]
    last_prop = jnp.sum(d * prop, axis=1, keepdims=True)
    props_ref[...] = jnp.where(m3, prop, jnp.broadcast_to(last_prop, (B, S, P)))


@jax.jit
def kernel(imgs, mask, W_bb, b_bb, W_c1, b_c1, W_c2, b_c2, W_d1, b_d1, W_d2, b_d2):
    x = imgs.reshape(B * S, D)
    BR = 1024
    NB = (B * S) // BR

    logp_f, prop_f, pred_f = pl.pallas_call(
        _dense_kernel,
        grid=(NB,),
        in_specs=[
            pl.BlockSpec((BR, D), lambda i: (i, 0)),
            pl.BlockSpec((D, NC), lambda i: (0, 0)),
            pl.BlockSpec((1, NC), lambda i: (0, 0)),
            pl.BlockSpec((NC, HID), lambda i: (0, 0)),
            pl.BlockSpec((1, HID), lambda i: (0, 0)),
            pl.BlockSpec((HID, P), lambda i: (0, 0)),
            pl.BlockSpec((1, P), lambda i: (0, 0)),
            pl.BlockSpec((P, HID), lambda i: (0, 0)),
            pl.BlockSpec((1, HID), lambda i: (0, 0)),
        ],
        out_specs=[
            pl.BlockSpec((BR, NC), lambda i: (i, 0)),
            pl.BlockSpec((BR, P), lambda i: (i, 0)),
            pl.BlockSpec((BR, HID), lambda i: (i, 0)),
        ],
        out_shape=[
            jax.ShapeDtypeStruct((B * S, NC), jnp.float32),
            jax.ShapeDtypeStruct((B * S, P), jnp.float32),
            jax.ShapeDtypeStruct((B * S, HID), jnp.float32),
        ],
    )(x, W_bb, b_bb.reshape(1, NC), W_c1, b_c1.reshape(1, HID),
      W_c2, b_c2.reshape(1, P), W_d1[NS:], b_d1.reshape(1, HID))

    pre_b = pred_f.reshape(B, S, HID)
    lengths = jnp.sum(mask, axis=1).astype(jnp.int32)
    bd2_pad = jnp.concatenate(
        [b_d2.astype(jnp.float32), jnp.full((16 - NS,), -jnp.inf, jnp.float32)])

    sc_fn = functools.partial(
        pl.kernel,
        out_type=jax.ShapeDtypeStruct((B, S, 128), jnp.float32),
        mesh=plsc.VectorSubcoreMesh(core_axis_name="c", subcore_axis_name="s"),
        compiler_params=pltpu.CompilerParams(needs_layout_passes=False),
        scratch_types=[
            pltpu.VMEM((S * HID,), jnp.float32),
            pltpu.VMEM((NS * HID,), jnp.float32),
            pltpu.VMEM((NS * HID,), jnp.float32),
            pltpu.VMEM((16,), jnp.float32),
            pltpu.VMEM((B,), jnp.int32),
            pltpu.VMEM((S, 128), jnp.float32),
        ],
    )(_sc_scan)
    zraw = sc_fn(pre_b.reshape(B, S * HID), lengths,
                 W_d1[:NS].reshape(-1), W_d2.T.reshape(-1), bd2_pad)

    mf = mask.astype(jnp.float32)
    ls, label2, vars_out, props_out = pl.pallas_call(
        _post_kernel,
        out_shape=[
            jax.ShapeDtypeStruct((B, S + 1, NS), jnp.float32),
            jax.ShapeDtypeStruct((B, 1), jnp.float32),
            jax.ShapeDtypeStruct((B, S, NC), jnp.float32),
            jax.ShapeDtypeStruct((B, S, P), jnp.float32),
        ],
    )(zraw, mf, logp_f.reshape(B, S, NC), prop_f.reshape(B, S, P))

    return (vars_out, props_out, ls, label2.reshape(B))
